# baseline (speedup 1.0000x reference)
import functools

import jax
import jax.numpy as jnp
from jax import lax
from jax.experimental import pallas as pl
from jax.experimental.pallas import tpu as pltpu
from jax.experimental.pallas import tpu_sc as plsc

_DIM = 256
_NPOINT = 64
_R2 = 16.0
_NS = 8
_B = 8
_N = 16384
_BIG = 1 << 30


_NCH = 8
_CW = _N // _NCH


def _fps_body(coorT_ref, ids_ref, scx_ref, scy_ref, scz_ref, dists_ref):
    li64 = lax.broadcasted_iota(jnp.int32, (_B, _NPOINT), 1)
    boff = lax.broadcasted_iota(jnp.int32, (_B, 1), 0) * _N
    dists_ref[...] = jnp.full((_B, _N), 1e10, jnp.float32)

    def step(i, carry):
        far, ids, sx, sy, sz = carry
        px = jnp.zeros((_B, 1), jnp.float32)
        py = jnp.zeros((_B, 1), jnp.float32)
        pz = jnp.zeros((_B, 1), jnp.float32)
        for c in range(_NCH):
            s0 = c * _CW
            lic = lax.broadcasted_iota(jnp.int32, (_B, _CW), 1) + s0
            m = lic == far
            cxc = coorT_ref[:, 0, s0:s0 + _CW]
            cyc = coorT_ref[:, 1, s0:s0 + _CW]
            czc = coorT_ref[:, 2, s0:s0 + _CW]
            px = px + jnp.sum(jnp.where(m, cxc, 0.0), axis=1, keepdims=True)
            py = py + jnp.sum(jnp.where(m, cyc, 0.0), axis=1, keepdims=True)
            pz = pz + jnp.sum(jnp.where(m, czc, 0.0), axis=1, keepdims=True)
        sel = li64 == i
        ids = jnp.where(sel, far + boff, ids)
        sx = jnp.where(sel, jnp.broadcast_to(px, (_B, _NPOINT)), sx)
        sy = jnp.where(sel, jnp.broadcast_to(py, (_B, _NPOINT)), sy)
        sz = jnp.where(sel, jnp.broadcast_to(pz, (_B, _NPOINT)), sz)
        bmx = jnp.full((_B, 1), -1.0, jnp.float32)
        barg = jnp.full((_B, 1), _N, jnp.int32)
        for c in range(_NCH):
            s0 = c * _CW
            lic = lax.broadcasted_iota(jnp.int32, (_B, _CW), 1) + s0
            dx = coorT_ref[:, 0, s0:s0 + _CW] - px
            dy = coorT_ref[:, 1, s0:s0 + _CW] - py
            dz = coorT_ref[:, 2, s0:s0 + _CW] - pz
            d = (dx * dx + dy * dy) + dz * dz
            dc = jnp.minimum(dists_ref[:, s0:s0 + _CW], d)
            dists_ref[:, s0:s0 + _CW] = dc
            cmx = jnp.max(dc, axis=1, keepdims=True)
            carg = jnp.min(jnp.where(dc == cmx, lic, _N), axis=1,
                           keepdims=True)
            better = (cmx > bmx) | ((cmx == bmx) & (carg < barg))
            bmx = jnp.where(better, cmx, bmx)
            barg = jnp.where(better, carg, barg)
        return barg, ids, sx, sy, sz

    init = (
        jnp.zeros((_B, 1), jnp.int32),
        jnp.zeros((_B, _NPOINT), jnp.int32),
        jnp.zeros((_B, _NPOINT), jnp.float32),
        jnp.zeros((_B, _NPOINT), jnp.float32),
        jnp.zeros((_B, _NPOINT), jnp.float32),
    )
    _, ids, sx, sy, sz = lax.fori_loop(0, _NPOINT, step, init)
    ids_ref[...] = ids
    scx_ref[...] = sx
    scy_ref[...] = sy
    scz_ref[...] = sz


def _run_fps(coorT):
    shape = jax.ShapeDtypeStruct((_B, _NPOINT), jnp.float32)
    ishape = jax.ShapeDtypeStruct((_B, _NPOINT), jnp.int32)
    return pl.pallas_call(
        _fps_body,
        out_shape=(ishape, shape, shape, shape),
        scratch_shapes=[pltpu.VMEM((_B, _N), jnp.float32)],
    )(coorT)


def _bq_body(coorT_ref, packed_ref, sc_ref, ids_ref,
             nidx_ref, diffc_ref, meanif_ref, sif_ref):
    b = pl.program_id(0)
    cxr = coorT_ref[0, 0:1, :]
    cyr = coorT_ref[0, 1:2, :]
    czr = coorT_ref[0, 2:3, :]
    scx = sc_ref[0, :, 0:1]
    scy = sc_ref[0, :, 1:2]
    scz = sc_ref[0, :, 2:3]
    dx = scx - cxr
    dy = scy - cyr
    dz = scz - czr
    d2 = (dx * dx + dy * dy) + dz * dz
    mask = d2 < _R2
    li = lax.broadcasted_iota(jnp.int32, (_NPOINT, _N), 1)
    cnt = jnp.sum(mask.astype(jnp.int32), axis=1, keepdims=True)
    mi = jnp.where(mask, li, _BIG)
    w = jnp.zeros((_NPOINT, _N), jnp.float32)
    idxs = []
    for j in range(_NS):
        mn = jnp.min(mi, axis=1, keepdims=True)
        idxs.append(mn)
        eq = mi == mn
        w = w + jnp.where(eq & (mn < _BIG), 1.0, 0.0)
        if j + 1 < _NS:
            mi = jnp.where(eq, _BIG, mi)
    first = jnp.where(cnt > 0, idxs[0], 0)
    pad = (_NS - jnp.minimum(cnt, _NS)).astype(jnp.float32)
    w = w + jnp.where(li == first, pad, 0.0)
    goff = b * _N
    for j in range(_NS):
        idx_j = jnp.where(j < cnt, idxs[j], first)
        nidx_ref[0, :, j:j + 1] = idx_j + goff
    eighth = jnp.float32(1.0 / _NS)
    packed = packed_ref[0]
    m6 = jnp.dot(w, packed, preferred_element_type=jnp.float32) * eighth
    diffc_ref[0, :, 0:1] = m6[:, 0:1] - scx
    diffc_ref[0, :, 1:2] = m6[:, 1:2] - scy
    diffc_ref[0, :, 2:3] = m6[:, 2:3] - scz
    meanif_ref[0, :, 0:3] = m6[:, 3:6]
    ws = (li == (ids_ref[0] - goff)).astype(jnp.float32)
    s6 = jnp.dot(ws, packed, preferred_element_type=jnp.float32)
    sif_ref[0, :, 0:3] = s6[:, 3:6]


def _run_bq(coorT, packed, sample_coor, ids_col):
    spec3 = pl.BlockSpec((1, _NPOINT, 3), lambda b: (b, 0, 0))
    return pl.pallas_call(
        _bq_body,
        grid=(_B,),
        in_specs=[
            pl.BlockSpec((1, 3, _N), lambda b: (b, 0, 0)),
            pl.BlockSpec((1, _N, 8), lambda b: (b, 0, 0)),
            spec3,
            pl.BlockSpec((1, _NPOINT, 1), lambda b: (b, 0, 0)),
        ],
        out_specs=[
            pl.BlockSpec((1, _NPOINT, _NS), lambda b: (b, 0, 0)),
            spec3, spec3, spec3,
        ],
        out_shape=[
            jax.ShapeDtypeStruct((_B, _NPOINT, _NS), jnp.int32),
            jax.ShapeDtypeStruct((_B, _NPOINT, 3), jnp.float32),
            jax.ShapeDtypeStruct((_B, _NPOINT, 3), jnp.float32),
            jax.ShapeDtypeStruct((_B, _NPOINT, 3), jnp.float32),
        ],
    )(coorT, packed, sample_coor, ids_col)


_NWORK = 32
_S_PER_W = (_B * _NPOINT) // _NWORK
_ROWS_PER_W = _S_PER_W * _NS


def _sc_body(x_hbm, sidx_hbm, nidx_hbm, sx_out, gx_out,
             sidx_v, nidx_v, srows, nrows, pooled, sem1, sem2):
    wid = lax.axis_index("s") * 2 + lax.axis_index("c")
    sb = wid * _S_PER_W
    nb = wid * _ROWS_PER_W
    pltpu.sync_copy(sidx_hbm.at[pl.ds(sb, _S_PER_W)], sidx_v)
    pltpu.sync_copy(nidx_hbm.at[pl.ds(nb, _ROWS_PER_W)], nidx_v)
    c1 = pltpu.async_copy(x_hbm.at[nidx_v], nrows, sem1)
    c2 = pltpu.async_copy(x_hbm.at[sidx_v], srows, sem2)
    c1.wait()

    def pool_one(s, carry):
        base = s * _NS
        for c in range(_DIM // 16):
            sl = pl.ds(c * 16, 16)
            m = nrows[base, sl]
            for r in range(1, _NS):
                m = jnp.maximum(m, nrows[base + r, sl])
            pooled[s, sl] = m
        return carry

    lax.fori_loop(0, _S_PER_W, pool_one, 0)
    c2.wait()
    pltpu.sync_copy(pooled, gx_out.at[pl.ds(sb, _S_PER_W)])
    pltpu.sync_copy(srows, sx_out.at[pl.ds(sb, _S_PER_W)])


def _run_gather_pool(x2d, sidx, nidx):
    nsamp = _B * _NPOINT
    mesh = plsc.VectorSubcoreMesh(core_axis_name="c", subcore_axis_name="s")
    f = pl.kernel(
        _sc_body,
        out_type=(
            jax.ShapeDtypeStruct((nsamp, _DIM), jnp.float32),
            jax.ShapeDtypeStruct((nsamp, _DIM), jnp.float32),
        ),
        mesh=mesh,
        scratch_types=[
            pltpu.VMEM((_S_PER_W,), jnp.int32),
            pltpu.VMEM((_ROWS_PER_W,), jnp.int32),
            pltpu.VMEM((_S_PER_W, _DIM), jnp.float32),
            pltpu.VMEM((_ROWS_PER_W, _DIM), jnp.float32),
            pltpu.VMEM((_S_PER_W, _DIM), jnp.float32),
            pltpu.SemaphoreType.DMA,
            pltpu.SemaphoreType.DMA,
        ],
    )
    return f(x2d, sidx, nidx)


def _ln(v, g, bvec):
    mu = jnp.mean(v, axis=-1, keepdims=True)
    var = jnp.mean((v - mu) ** 2, axis=-1, keepdims=True)
    return (v - mu) / jnp.sqrt(var + 1e-5) * g + bvec


def _attn_body(sx_ref, gx_ref, vc_ref, vi_ref, sc_ref, sif_ref,
               wqt_ref, wkt_ref, gq_ref, bq_ref, gk_ref, bk_ref,
               outx_ref, outc_ref, outi_ref):
    sxb = sx_ref[0]
    gxb = gx_ref[0]
    x2 = gxb - sxb
    nk = _ln(sxb, gk_ref[...], bk_ref[...])
    nq = _ln(x2, gq_ref[...], bq_ref[...])
    qh = jnp.dot(nq, wqt_ref[...], preferred_element_type=jnp.float32)
    kh = jnp.dot(nk, wkt_ref[...], preferred_element_type=jnp.float32)
    attn = lax.dot_general(qh, kh, (((1,), (1,)), ((), ())),
                           preferred_element_type=jnp.float32)
    mx = jnp.max(attn, axis=-1, keepdims=True)
    e = jnp.exp(attn - mx)
    p = e / jnp.sum(e, axis=-1, keepdims=True)
    c2 = jnp.dot(p, vc_ref[0], preferred_element_type=jnp.float32)
    i2 = jnp.dot(p, vi_ref[0], preferred_element_type=jnp.float32)
    outx_ref[0] = sxb + x2
    outc_ref[0] = sc_ref[0] + c2
    outi_ref[0] = sif_ref[0] + i2


def _run_attn(sx, gx, v_c, v_i, sample_coor, sif, WqT, WkT, gq, bq, gk, bk):
    spec64 = pl.BlockSpec((1, _NPOINT, _DIM), lambda b: (b, 0, 0))
    spec3 = pl.BlockSpec((1, _NPOINT, 3), lambda b: (b, 0, 0))
    specw = pl.BlockSpec((_DIM, _DIM), lambda b: (0, 0))
    specv = pl.BlockSpec((1, _DIM), lambda b: (0, 0))
    return pl.pallas_call(
        _attn_body,
        grid=(_B,),
        in_specs=[spec64, spec64, spec3, spec3, spec3, spec3,
                  specw, specw, specv, specv, specv, specv],
        out_specs=[spec64, spec3, spec3],
        out_shape=[
            jax.ShapeDtypeStruct((_B, _NPOINT, _DIM), jnp.float32),
            jax.ShapeDtypeStruct((_B, _NPOINT, 3), jnp.float32),
            jax.ShapeDtypeStruct((_B, _NPOINT, 3), jnp.float32),
        ],
    )(sx, gx, v_c, v_i, sample_coor, sif, WqT, WkT, gq, bq, gk, bk)


def kernel(input_feature, x, coor, Wq, Wk, gq, bq, gk, bk):
    coorT = jnp.transpose(coor, (0, 2, 1))
    packed = jnp.concatenate(
        [coor, input_feature, jnp.zeros((_B, _N, 2), jnp.float32)], axis=-1)
    ids_g, scx, scy, scz = _run_fps(coorT)
    sample_coor = jnp.stack([scx, scy, scz], axis=-1)
    nidx, diffc, meanif, sif = _run_bq(
        coorT, packed, sample_coor, ids_g.reshape(_B, _NPOINT, 1))
    sx, gx = _run_gather_pool(
        x.reshape(_B * _N, _DIM), ids_g.reshape(-1), nidx.reshape(-1))
    v_c = jnp.transpose(diffc, (0, 2, 1)).reshape(_B, _NPOINT, 3)
    v_i = jnp.transpose(meanif, (0, 2, 1)).reshape(_B, _NPOINT, 3)
    return _run_attn(
        sx.reshape(_B, _NPOINT, _DIM), gx.reshape(_B, _NPOINT, _DIM),
        v_c, v_i, sample_coor, sif, Wq.T, Wk.T,
        gq.reshape(1, _DIM), bq.reshape(1, _DIM),
        gk.reshape(1, _DIM), bk.reshape(1, _DIM))


# speedup vs baseline: 1.5513x; 1.5513x over previous
import functools

import jax
import jax.numpy as jnp
from jax import lax
from jax.experimental import pallas as pl
from jax.experimental.pallas import tpu as pltpu
from jax.experimental.pallas import tpu_sc as plsc

_DIM = 256
_NPOINT = 64
_R2 = 16.0
_NS = 8
_B = 8
_N = 16384
_BIG = 1 << 30


_NCH = 8
_CW = _N // _NCH


def _fps_body(coorT_ref, ids_ref, scx_ref, scy_ref, scz_ref, dists_ref):
    li64 = lax.broadcasted_iota(jnp.int32, (_B, _NPOINT), 1)
    boff = lax.broadcasted_iota(jnp.int32, (_B, 1), 0) * _N
    dists_ref[...] = jnp.full((_B, _N), 1e10, jnp.float32)

    def step(i, carry):
        far, ids, sx, sy, sz = carry
        px = jnp.zeros((_B, 1), jnp.float32)
        py = jnp.zeros((_B, 1), jnp.float32)
        pz = jnp.zeros((_B, 1), jnp.float32)
        for c in range(_NCH):
            s0 = c * _CW
            lic = lax.broadcasted_iota(jnp.int32, (_B, _CW), 1) + s0
            m = lic == far
            cxc = coorT_ref[:, 0, s0:s0 + _CW]
            cyc = coorT_ref[:, 1, s0:s0 + _CW]
            czc = coorT_ref[:, 2, s0:s0 + _CW]
            px = px + jnp.sum(jnp.where(m, cxc, 0.0), axis=1, keepdims=True)
            py = py + jnp.sum(jnp.where(m, cyc, 0.0), axis=1, keepdims=True)
            pz = pz + jnp.sum(jnp.where(m, czc, 0.0), axis=1, keepdims=True)
        sel = li64 == i
        ids = jnp.where(sel, far + boff, ids)
        sx = jnp.where(sel, jnp.broadcast_to(px, (_B, _NPOINT)), sx)
        sy = jnp.where(sel, jnp.broadcast_to(py, (_B, _NPOINT)), sy)
        sz = jnp.where(sel, jnp.broadcast_to(pz, (_B, _NPOINT)), sz)
        bmx = jnp.full((_B, 1), -1.0, jnp.float32)
        barg = jnp.full((_B, 1), _N, jnp.int32)
        for c in range(_NCH):
            s0 = c * _CW
            lic = lax.broadcasted_iota(jnp.int32, (_B, _CW), 1) + s0
            dx = coorT_ref[:, 0, s0:s0 + _CW] - px
            dy = coorT_ref[:, 1, s0:s0 + _CW] - py
            dz = coorT_ref[:, 2, s0:s0 + _CW] - pz
            d = (dx * dx + dy * dy) + dz * dz
            dc = jnp.minimum(dists_ref[:, s0:s0 + _CW], d)
            dists_ref[:, s0:s0 + _CW] = dc
            cmx = jnp.max(dc, axis=1, keepdims=True)
            carg = jnp.min(jnp.where(dc == cmx, lic, _N), axis=1,
                           keepdims=True)
            better = (cmx > bmx) | ((cmx == bmx) & (carg < barg))
            bmx = jnp.where(better, cmx, bmx)
            barg = jnp.where(better, carg, barg)
        return barg, ids, sx, sy, sz

    init = (
        jnp.zeros((_B, 1), jnp.int32),
        jnp.zeros((_B, _NPOINT), jnp.int32),
        jnp.zeros((_B, _NPOINT), jnp.float32),
        jnp.zeros((_B, _NPOINT), jnp.float32),
        jnp.zeros((_B, _NPOINT), jnp.float32),
    )
    _, ids, sx, sy, sz = lax.fori_loop(0, _NPOINT, step, init)
    ids_ref[...] = ids
    scx_ref[...] = sx
    scy_ref[...] = sy
    scz_ref[...] = sz


def _run_fps(coorT):
    shape = jax.ShapeDtypeStruct((_B, _NPOINT), jnp.float32)
    ishape = jax.ShapeDtypeStruct((_B, _NPOINT), jnp.int32)
    return pl.pallas_call(
        _fps_body,
        out_shape=(ishape, shape, shape, shape),
        scratch_shapes=[pltpu.VMEM((_B, _N), jnp.float32)],
    )(coorT)


def _bq_body(packedT_ref, sc_ref, scT_ref, ids_ref,
             nidx_ref, diffc_ref, meanif_ref, sif_ref):
    b = pl.program_id(0)
    cxr = packedT_ref[0, 0:1, :]
    cyr = packedT_ref[0, 1:2, :]
    czr = packedT_ref[0, 2:3, :]
    scx = sc_ref[0, :, 0:1]
    scy = sc_ref[0, :, 1:2]
    scz = sc_ref[0, :, 2:3]
    dx = scx - cxr
    dy = scy - cyr
    dz = scz - czr
    d2 = (dx * dx + dy * dy) + dz * dz
    mask = d2 < _R2
    li = lax.broadcasted_iota(jnp.int32, (_NPOINT, _N), 1)
    cnt = jnp.sum(mask.astype(jnp.int32), axis=1, keepdims=True)
    mi = jnp.where(mask, li, _BIG)
    w = jnp.zeros((_NPOINT, _N), jnp.float32)
    idxs = []
    for j in range(_NS):
        mn = jnp.min(mi, axis=1, keepdims=True)
        idxs.append(mn)
        eq = mi == mn
        w = w + jnp.where(eq & (mn < _BIG), 1.0, 0.0)
        if j + 1 < _NS:
            mi = jnp.where(eq, _BIG, mi)
    first = jnp.where(cnt > 0, idxs[0], 0)
    pad = (_NS - jnp.minimum(cnt, _NS)).astype(jnp.float32)
    w = w + jnp.where(li == first, pad, 0.0)
    goff = b * _N
    for j in range(_NS):
        idx_j = jnp.where(j < cnt, idxs[j], first)
        nidx_ref[0, :, j:j + 1] = idx_j + goff
    eighth = jnp.float32(1.0 / _NS)
    p8 = packedT_ref[0]
    m6 = lax.dot_general(p8, w, (((1,), (1,)), ((), ())),
                         preferred_element_type=jnp.float32) * eighth
    diffc_ref[0, 0:3, :] = m6[0:3, :] - scT_ref[0]
    meanif_ref[0, 0:3, :] = m6[3:6, :]
    ws = (li == (ids_ref[0] - goff)).astype(jnp.float32)
    s6 = lax.dot_general(p8, ws, (((1,), (1,)), ((), ())),
                         preferred_element_type=jnp.float32)
    sif_ref[0, 0:3, :] = s6[3:6, :]


def _run_bq(packedT, sample_coor, scT, ids_col):
    spec3c = pl.BlockSpec((1, 3, _NPOINT), lambda b: (b, 0, 0))
    return pl.pallas_call(
        _bq_body,
        grid=(_B,),
        in_specs=[
            pl.BlockSpec((1, 8, _N), lambda b: (b, 0, 0)),
            pl.BlockSpec((1, _NPOINT, 3), lambda b: (b, 0, 0)),
            spec3c,
            pl.BlockSpec((1, _NPOINT, 1), lambda b: (b, 0, 0)),
        ],
        out_specs=[
            pl.BlockSpec((1, _NPOINT, _NS), lambda b: (b, 0, 0)),
            spec3c, spec3c, spec3c,
        ],
        out_shape=[
            jax.ShapeDtypeStruct((_B, _NPOINT, _NS), jnp.int32),
            jax.ShapeDtypeStruct((_B, 3, _NPOINT), jnp.float32),
            jax.ShapeDtypeStruct((_B, 3, _NPOINT), jnp.float32),
            jax.ShapeDtypeStruct((_B, 3, _NPOINT), jnp.float32),
        ],
    )(packedT, sample_coor, scT, ids_col)


_NWORK = 32
_S_PER_W = (_B * _NPOINT) // _NWORK
_ROWS_PER_W = _S_PER_W * _NS


def _sc_body(x_hbm, sidx_hbm, nidx_hbm, sx_out, gx_out,
             sidx_v, nidx_v, srows, nrows, pooled, sem1, sem2):
    wid = lax.axis_index("s") * 2 + lax.axis_index("c")
    sb = wid * _S_PER_W
    nb = wid * _ROWS_PER_W
    pltpu.sync_copy(sidx_hbm.at[pl.ds(sb, _S_PER_W)], sidx_v)
    pltpu.sync_copy(nidx_hbm.at[pl.ds(nb, _ROWS_PER_W)], nidx_v)
    c1 = pltpu.async_copy(x_hbm.at[nidx_v], nrows, sem1)
    c2 = pltpu.async_copy(x_hbm.at[sidx_v], srows, sem2)
    c1.wait()

    def pool_one(s, carry):
        base = s * _NS
        for c in range(_DIM // 16):
            sl = pl.ds(c * 16, 16)
            m = nrows[base, sl]
            for r in range(1, _NS):
                m = jnp.maximum(m, nrows[base + r, sl])
            pooled[s, sl] = m
        return carry

    lax.fori_loop(0, _S_PER_W, pool_one, 0)
    c2.wait()
    pltpu.sync_copy(pooled, gx_out.at[pl.ds(sb, _S_PER_W)])
    pltpu.sync_copy(srows, sx_out.at[pl.ds(sb, _S_PER_W)])


def _run_gather_pool(x2d, sidx, nidx):
    nsamp = _B * _NPOINT
    mesh = plsc.VectorSubcoreMesh(core_axis_name="c", subcore_axis_name="s")
    f = pl.kernel(
        _sc_body,
        out_type=(
            jax.ShapeDtypeStruct((nsamp, _DIM), jnp.float32),
            jax.ShapeDtypeStruct((nsamp, _DIM), jnp.float32),
        ),
        mesh=mesh,
        scratch_types=[
            pltpu.VMEM((_S_PER_W,), jnp.int32),
            pltpu.VMEM((_ROWS_PER_W,), jnp.int32),
            pltpu.VMEM((_S_PER_W, _DIM), jnp.float32),
            pltpu.VMEM((_ROWS_PER_W, _DIM), jnp.float32),
            pltpu.VMEM((_S_PER_W, _DIM), jnp.float32),
            pltpu.SemaphoreType.DMA,
            pltpu.SemaphoreType.DMA,
        ],
    )
    return f(x2d, sidx, nidx)


def _ln(v, g, bvec):
    mu = jnp.mean(v, axis=-1, keepdims=True)
    var = jnp.mean((v - mu) ** 2, axis=-1, keepdims=True)
    return (v - mu) / jnp.sqrt(var + 1e-5) * g + bvec


def _attn_body(sx_ref, gx_ref, vc_ref, vi_ref, sc_ref, sif_ref,
               wqt_ref, wkt_ref, gq_ref, bq_ref, gk_ref, bk_ref,
               outx_ref, outc_ref, outi_ref):
    sxb = sx_ref[0]
    gxb = gx_ref[0]
    x2 = gxb - sxb
    nk = _ln(sxb, gk_ref[...], bk_ref[...])
    nq = _ln(x2, gq_ref[...], bq_ref[...])
    qh = jnp.dot(nq, wqt_ref[...], preferred_element_type=jnp.float32)
    kh = jnp.dot(nk, wkt_ref[...], preferred_element_type=jnp.float32)
    attn = lax.dot_general(qh, kh, (((1,), (1,)), ((), ())),
                           preferred_element_type=jnp.float32)
    mx = jnp.max(attn, axis=-1, keepdims=True)
    e = jnp.exp(attn - mx)
    p = e / jnp.sum(e, axis=-1, keepdims=True)
    c2 = jnp.dot(p, vc_ref[0], preferred_element_type=jnp.float32)
    i2 = jnp.dot(p, vi_ref[0], preferred_element_type=jnp.float32)
    outx_ref[0] = sxb + x2
    outc_ref[0] = sc_ref[0] + c2
    outi_ref[0] = sif_ref[0] + i2


def _run_attn(sx, gx, v_c, v_i, sample_coor, sif, WqT, WkT, gq, bq, gk, bk):
    spec64 = pl.BlockSpec((1, _NPOINT, _DIM), lambda b: (b, 0, 0))
    spec3 = pl.BlockSpec((1, _NPOINT, 3), lambda b: (b, 0, 0))
    specw = pl.BlockSpec((_DIM, _DIM), lambda b: (0, 0))
    specv = pl.BlockSpec((1, _DIM), lambda b: (0, 0))
    return pl.pallas_call(
        _attn_body,
        grid=(_B,),
        in_specs=[spec64, spec64, spec3, spec3, spec3, spec3,
                  specw, specw, specv, specv, specv, specv],
        out_specs=[spec64, spec3, spec3],
        out_shape=[
            jax.ShapeDtypeStruct((_B, _NPOINT, _DIM), jnp.float32),
            jax.ShapeDtypeStruct((_B, _NPOINT, 3), jnp.float32),
            jax.ShapeDtypeStruct((_B, _NPOINT, 3), jnp.float32),
        ],
    )(sx, gx, v_c, v_i, sample_coor, sif, WqT, WkT, gq, bq, gk, bk)


def kernel(input_feature, x, coor, Wq, Wk, gq, bq, gk, bk):
    coorT = jnp.transpose(coor, (0, 2, 1))
    ifT = jnp.transpose(input_feature, (0, 2, 1))
    packedT = jnp.concatenate(
        [coorT, ifT, jnp.zeros((_B, 2, _N), jnp.float32)], axis=1)
    ids_g, scx, scy, scz = _run_fps(coorT)
    sample_coor = jnp.stack([scx, scy, scz], axis=-1)
    scT = jnp.stack([scx, scy, scz], axis=1)
    nidx, diffc, meanif, sif3 = _run_bq(
        packedT, sample_coor, scT, ids_g.reshape(_B, _NPOINT, 1))
    sx, gx = _run_gather_pool(
        x.reshape(_B * _N, _DIM), ids_g.reshape(-1), nidx.reshape(-1))
    v_c = diffc.reshape(_B, _NPOINT, 3)
    v_i = meanif.reshape(_B, _NPOINT, 3)
    sif = jnp.transpose(sif3, (0, 2, 1))
    return _run_attn(
        sx.reshape(_B, _NPOINT, _DIM), gx.reshape(_B, _NPOINT, _DIM),
        v_c, v_i, sample_coor, sif, Wq.T, Wk.T,
        gq.reshape(1, _DIM), bq.reshape(1, _DIM),
        gk.reshape(1, _DIM), bk.reshape(1, _DIM))


# speedup vs baseline: 1.7835x; 1.1497x over previous
import functools

import jax
import jax.numpy as jnp
from jax import lax
from jax.experimental import pallas as pl
from jax.experimental.pallas import tpu as pltpu
from jax.experimental.pallas import tpu_sc as plsc

_DIM = 256
_NPOINT = 64
_R2 = 16.0
_NS = 8
_B = 8
_N = 16384
_BIG = 1 << 30


_NCH = 8
_CW = _N // _NCH


def _fps_body(coorT_ref, ids_ref, scx_ref, scy_ref, scz_ref, dists_ref):
    li64 = lax.broadcasted_iota(jnp.int32, (_B, _NPOINT), 1)
    boff = lax.broadcasted_iota(jnp.int32, (_B, 1), 0) * _N
    dists_ref[...] = jnp.full((_B, _N), 1e10, jnp.float32)

    def step(i, carry):
        far, ids, sx, sy, sz = carry
        px = jnp.zeros((_B, 1), jnp.float32)
        py = jnp.zeros((_B, 1), jnp.float32)
        pz = jnp.zeros((_B, 1), jnp.float32)
        for c in range(_NCH):
            s0 = c * _CW
            lic = lax.broadcasted_iota(jnp.int32, (_B, _CW), 1) + s0
            m = lic == far
            cxc = coorT_ref[:, 0, s0:s0 + _CW]
            cyc = coorT_ref[:, 1, s0:s0 + _CW]
            czc = coorT_ref[:, 2, s0:s0 + _CW]
            px = px + jnp.sum(jnp.where(m, cxc, 0.0), axis=1, keepdims=True)
            py = py + jnp.sum(jnp.where(m, cyc, 0.0), axis=1, keepdims=True)
            pz = pz + jnp.sum(jnp.where(m, czc, 0.0), axis=1, keepdims=True)
        sel = li64 == i
        ids = jnp.where(sel, far + boff, ids)
        sx = jnp.where(sel, jnp.broadcast_to(px, (_B, _NPOINT)), sx)
        sy = jnp.where(sel, jnp.broadcast_to(py, (_B, _NPOINT)), sy)
        sz = jnp.where(sel, jnp.broadcast_to(pz, (_B, _NPOINT)), sz)
        bmx = jnp.full((_B, 1), -1.0, jnp.float32)
        barg = jnp.full((_B, 1), _N, jnp.int32)
        for c in range(_NCH):
            s0 = c * _CW
            lic = lax.broadcasted_iota(jnp.int32, (_B, _CW), 1) + s0
            dx = coorT_ref[:, 0, s0:s0 + _CW] - px
            dy = coorT_ref[:, 1, s0:s0 + _CW] - py
            dz = coorT_ref[:, 2, s0:s0 + _CW] - pz
            d = (dx * dx + dy * dy) + dz * dz
            dc = jnp.minimum(dists_ref[:, s0:s0 + _CW], d)
            dists_ref[:, s0:s0 + _CW] = dc
            cmx = jnp.max(dc, axis=1, keepdims=True)
            carg = jnp.min(jnp.where(dc == cmx, lic, _N), axis=1,
                           keepdims=True)
            better = (cmx > bmx) | ((cmx == bmx) & (carg < barg))
            bmx = jnp.where(better, cmx, bmx)
            barg = jnp.where(better, carg, barg)
        return barg, ids, sx, sy, sz

    init = (
        jnp.zeros((_B, 1), jnp.int32),
        jnp.zeros((_B, _NPOINT), jnp.int32),
        jnp.zeros((_B, _NPOINT), jnp.float32),
        jnp.zeros((_B, _NPOINT), jnp.float32),
        jnp.zeros((_B, _NPOINT), jnp.float32),
    )
    _, ids, sx, sy, sz = lax.fori_loop(0, _NPOINT, step, init)
    ids_ref[...] = ids
    scx_ref[...] = sx
    scy_ref[...] = sy
    scz_ref[...] = sz


def _run_fps(coorT):
    shape = jax.ShapeDtypeStruct((_B, _NPOINT), jnp.float32)
    ishape = jax.ShapeDtypeStruct((_B, _NPOINT), jnp.int32)
    return pl.pallas_call(
        _fps_body,
        out_shape=(ishape, shape, shape, shape),
        scratch_shapes=[pltpu.VMEM((_B, _N), jnp.float32)],
    )(coorT)


def _bq_body(packedT_ref, sc_ref, scT_ref, ids_ref,
             nidx_ref, diffc_ref, meanif_ref, sif_ref):
    b = pl.program_id(0)
    cxr = packedT_ref[0, 0:1, :]
    cyr = packedT_ref[0, 1:2, :]
    czr = packedT_ref[0, 2:3, :]
    scx = sc_ref[0, :, 0:1]
    scy = sc_ref[0, :, 1:2]
    scz = sc_ref[0, :, 2:3]
    dx = scx - cxr
    dy = scy - cyr
    dz = scz - czr
    d2 = (dx * dx + dy * dy) + dz * dz
    mask = d2 < _R2
    li = lax.broadcasted_iota(jnp.int32, (_NPOINT, _N), 1)
    cnt = jnp.sum(mask.astype(jnp.int32), axis=1, keepdims=True)
    mi0 = jnp.where(mask, li, _BIG)
    mi = mi0
    idxs = []
    for j in range(_NS):
        mn = jnp.min(mi, axis=1, keepdims=True)
        idxs.append(mn)
        mi = jnp.where(mi == mn, _BIG, mi)
    w = jnp.where(mi != mi0, 1.0, 0.0)
    first = jnp.where(cnt > 0, idxs[0], 0)
    pad = (_NS - jnp.minimum(cnt, _NS)).astype(jnp.float32)
    w = w + jnp.where(li == first, pad, 0.0)
    goff = b * _N
    for j in range(_NS):
        idx_j = jnp.where(j < cnt, idxs[j], first)
        nidx_ref[0, :, j:j + 1] = idx_j + goff
    eighth = jnp.float32(1.0 / _NS)
    p8 = packedT_ref[0]
    m6 = lax.dot_general(p8, w, (((1,), (1,)), ((), ())),
                         preferred_element_type=jnp.float32) * eighth
    diffc_ref[0, 0:3, :] = m6[0:3, :] - scT_ref[0]
    meanif_ref[0, 0:3, :] = m6[3:6, :]
    ws = (li == (ids_ref[0] - goff)).astype(jnp.float32)
    s6 = lax.dot_general(p8, ws, (((1,), (1,)), ((), ())),
                         preferred_element_type=jnp.float32)
    sif_ref[0, 0:3, :] = s6[3:6, :]


def _run_bq(packedT, sample_coor, scT, ids_col):
    spec3c = pl.BlockSpec((1, 3, _NPOINT), lambda b: (b, 0, 0))
    return pl.pallas_call(
        _bq_body,
        grid=(_B,),
        in_specs=[
            pl.BlockSpec((1, 8, _N), lambda b: (b, 0, 0)),
            pl.BlockSpec((1, _NPOINT, 3), lambda b: (b, 0, 0)),
            spec3c,
            pl.BlockSpec((1, _NPOINT, 1), lambda b: (b, 0, 0)),
        ],
        out_specs=[
            pl.BlockSpec((1, _NPOINT, _NS), lambda b: (b, 0, 0)),
            spec3c, spec3c, spec3c,
        ],
        out_shape=[
            jax.ShapeDtypeStruct((_B, _NPOINT, _NS), jnp.int32),
            jax.ShapeDtypeStruct((_B, 3, _NPOINT), jnp.float32),
            jax.ShapeDtypeStruct((_B, 3, _NPOINT), jnp.float32),
            jax.ShapeDtypeStruct((_B, 3, _NPOINT), jnp.float32),
        ],
    )(packedT, sample_coor, scT, ids_col)


_NWORK = 32
_S_PER_W = (_B * _NPOINT) // _NWORK
_ROWS_PER_W = _S_PER_W * _NS


def _sc_body(x_hbm, sidx_hbm, nidx_hbm, sx_out, gx_out,
             sidx_v, nidx_v, srows, nrows, pooled, sem1, sem2):
    wid = lax.axis_index("s") * 2 + lax.axis_index("c")
    sb = wid * _S_PER_W
    nb = wid * _ROWS_PER_W
    pltpu.sync_copy(sidx_hbm.at[pl.ds(sb, _S_PER_W)], sidx_v)
    pltpu.sync_copy(nidx_hbm.at[pl.ds(nb, _ROWS_PER_W)], nidx_v)
    c1 = pltpu.async_copy(x_hbm.at[nidx_v], nrows, sem1)
    c2 = pltpu.async_copy(x_hbm.at[sidx_v], srows, sem2)
    c1.wait()

    def pool_one(s, carry):
        base = s * _NS
        for c in range(_DIM // 16):
            sl = pl.ds(c * 16, 16)
            m = nrows[base, sl]
            for r in range(1, _NS):
                m = jnp.maximum(m, nrows[base + r, sl])
            pooled[s, sl] = m
        return carry

    lax.fori_loop(0, _S_PER_W, pool_one, 0)
    c2.wait()
    pltpu.sync_copy(pooled, gx_out.at[pl.ds(sb, _S_PER_W)])
    pltpu.sync_copy(srows, sx_out.at[pl.ds(sb, _S_PER_W)])


def _run_gather_pool(x2d, sidx, nidx):
    nsamp = _B * _NPOINT
    mesh = plsc.VectorSubcoreMesh(core_axis_name="c", subcore_axis_name="s")
    f = pl.kernel(
        _sc_body,
        out_type=(
            jax.ShapeDtypeStruct((nsamp, _DIM), jnp.float32),
            jax.ShapeDtypeStruct((nsamp, _DIM), jnp.float32),
        ),
        mesh=mesh,
        scratch_types=[
            pltpu.VMEM((_S_PER_W,), jnp.int32),
            pltpu.VMEM((_ROWS_PER_W,), jnp.int32),
            pltpu.VMEM((_S_PER_W, _DIM), jnp.float32),
            pltpu.VMEM((_ROWS_PER_W, _DIM), jnp.float32),
            pltpu.VMEM((_S_PER_W, _DIM), jnp.float32),
            pltpu.SemaphoreType.DMA,
            pltpu.SemaphoreType.DMA,
        ],
    )
    return f(x2d, sidx, nidx)


def _ln(v, g, bvec):
    mu = jnp.mean(v, axis=-1, keepdims=True)
    var = jnp.mean((v - mu) ** 2, axis=-1, keepdims=True)
    return (v - mu) / jnp.sqrt(var + 1e-5) * g + bvec


def _attn_body(sx_ref, gx_ref, vc_ref, vi_ref, sc_ref, sif_ref,
               wqt_ref, wkt_ref, gq_ref, bq_ref, gk_ref, bk_ref,
               outx_ref, outc_ref, outi_ref):
    sxb = sx_ref[0]
    gxb = gx_ref[0]
    x2 = gxb - sxb
    nk = _ln(sxb, gk_ref[...], bk_ref[...])
    nq = _ln(x2, gq_ref[...], bq_ref[...])
    qh = jnp.dot(nq, wqt_ref[...], preferred_element_type=jnp.float32)
    kh = jnp.dot(nk, wkt_ref[...], preferred_element_type=jnp.float32)
    attn = lax.dot_general(qh, kh, (((1,), (1,)), ((), ())),
                           preferred_element_type=jnp.float32)
    mx = jnp.max(attn, axis=-1, keepdims=True)
    e = jnp.exp(attn - mx)
    p = e / jnp.sum(e, axis=-1, keepdims=True)
    c2 = jnp.dot(p, vc_ref[0], preferred_element_type=jnp.float32)
    i2 = jnp.dot(p, vi_ref[0], preferred_element_type=jnp.float32)
    outx_ref[0] = sxb + x2
    outc_ref[0] = sc_ref[0] + c2
    outi_ref[0] = sif_ref[0] + i2


def _run_attn(sx, gx, v_c, v_i, sample_coor, sif, WqT, WkT, gq, bq, gk, bk):
    spec64 = pl.BlockSpec((1, _NPOINT, _DIM), lambda b: (b, 0, 0))
    spec3 = pl.BlockSpec((1, _NPOINT, 3), lambda b: (b, 0, 0))
    specw = pl.BlockSpec((_DIM, _DIM), lambda b: (0, 0))
    specv = pl.BlockSpec((1, _DIM), lambda b: (0, 0))
    return pl.pallas_call(
        _attn_body,
        grid=(_B,),
        in_specs=[spec64, spec64, spec3, spec3, spec3, spec3,
                  specw, specw, specv, specv, specv, specv],
        out_specs=[spec64, spec3, spec3],
        out_shape=[
            jax.ShapeDtypeStruct((_B, _NPOINT, _DIM), jnp.float32),
            jax.ShapeDtypeStruct((_B, _NPOINT, 3), jnp.float32),
            jax.ShapeDtypeStruct((_B, _NPOINT, 3), jnp.float32),
        ],
    )(sx, gx, v_c, v_i, sample_coor, sif, WqT, WkT, gq, bq, gk, bk)


def kernel(input_feature, x, coor, Wq, Wk, gq, bq, gk, bk):
    coorT = jnp.transpose(coor, (0, 2, 1))
    ifT = jnp.transpose(input_feature, (0, 2, 1))
    packedT = jnp.concatenate(
        [coorT, ifT, jnp.zeros((_B, 2, _N), jnp.float32)], axis=1)
    ids_g, scx, scy, scz = _run_fps(coorT)
    sample_coor = jnp.stack([scx, scy, scz], axis=-1)
    scT = jnp.stack([scx, scy, scz], axis=1)
    nidx, diffc, meanif, sif3 = _run_bq(
        packedT, sample_coor, scT, ids_g.reshape(_B, _NPOINT, 1))
    sx, gx = _run_gather_pool(
        x.reshape(_B * _N, _DIM), ids_g.reshape(-1), nidx.reshape(-1))
    v_c = diffc.reshape(_B, _NPOINT, 3)
    v_i = meanif.reshape(_B, _NPOINT, 3)
    sif = jnp.transpose(sif3, (0, 2, 1))
    return _run_attn(
        sx.reshape(_B, _NPOINT, _DIM), gx.reshape(_B, _NPOINT, _DIM),
        v_c, v_i, sample_coor, sif, Wq.T, Wk.T,
        gq.reshape(1, _DIM), bq.reshape(1, _DIM),
        gk.reshape(1, _DIM), bk.reshape(1, _DIM))


# speedup vs baseline: 1.7838x; 1.0001x over previous
import jax
import jax.numpy as jnp
from jax import lax
from jax.experimental import pallas as pl
from jax.experimental.pallas import tpu as pltpu
from jax.experimental.pallas import tpu_sc as plsc

_DIM = 256
_NPOINT = 64
_R2 = 16.0
_NS = 8
_B = 8
_N = 16384
_BIG = 1 << 30


_NCH = 8
_CW = _N // _NCH


def _fps_body(coorT_ref, ids_ref, scx_ref, scy_ref, scz_ref, dists_ref):
    li64 = lax.broadcasted_iota(jnp.int32, (_B, _NPOINT), 1)
    boff = lax.broadcasted_iota(jnp.int32, (_B, 1), 0) * _N
    dists_ref[...] = jnp.full((_B, _N), 1e10, jnp.float32)

    def step(i, carry):
        far, ids, sx, sy, sz = carry
        px = jnp.zeros((_B, 1), jnp.float32)
        py = jnp.zeros((_B, 1), jnp.float32)
        pz = jnp.zeros((_B, 1), jnp.float32)
        for c in range(_NCH):
            s0 = c * _CW
            lic = lax.broadcasted_iota(jnp.int32, (_B, _CW), 1) + s0
            m = lic == far
            cxc = coorT_ref[:, 0, s0:s0 + _CW]
            cyc = coorT_ref[:, 1, s0:s0 + _CW]
            czc = coorT_ref[:, 2, s0:s0 + _CW]
            px = px + jnp.sum(jnp.where(m, cxc, 0.0), axis=1, keepdims=True)
            py = py + jnp.sum(jnp.where(m, cyc, 0.0), axis=1, keepdims=True)
            pz = pz + jnp.sum(jnp.where(m, czc, 0.0), axis=1, keepdims=True)
        sel = li64 == i
        ids = jnp.where(sel, far + boff, ids)
        sx = jnp.where(sel, jnp.broadcast_to(px, (_B, _NPOINT)), sx)
        sy = jnp.where(sel, jnp.broadcast_to(py, (_B, _NPOINT)), sy)
        sz = jnp.where(sel, jnp.broadcast_to(pz, (_B, _NPOINT)), sz)
        bmx = jnp.full((_B, 1), -1.0, jnp.float32)
        barg = jnp.full((_B, 1), _N, jnp.int32)
        for c in range(_NCH):
            s0 = c * _CW
            lic = lax.broadcasted_iota(jnp.int32, (_B, _CW), 1) + s0
            dx = coorT_ref[:, 0, s0:s0 + _CW] - px
            dy = coorT_ref[:, 1, s0:s0 + _CW] - py
            dz = coorT_ref[:, 2, s0:s0 + _CW] - pz
            d = (dx * dx + dy * dy) + dz * dz
            dc = jnp.minimum(dists_ref[:, s0:s0 + _CW], d)
            dists_ref[:, s0:s0 + _CW] = dc
            cmx = jnp.max(dc, axis=1, keepdims=True)
            carg = jnp.min(jnp.where(dc == cmx, lic, _N), axis=1,
                           keepdims=True)
            better = (cmx > bmx) | ((cmx == bmx) & (carg < barg))
            bmx = jnp.where(better, cmx, bmx)
            barg = jnp.where(better, carg, barg)
        return barg, ids, sx, sy, sz

    init = (
        jnp.zeros((_B, 1), jnp.int32),
        jnp.zeros((_B, _NPOINT), jnp.int32),
        jnp.zeros((_B, _NPOINT), jnp.float32),
        jnp.zeros((_B, _NPOINT), jnp.float32),
        jnp.zeros((_B, _NPOINT), jnp.float32),
    )
    _, ids, sx, sy, sz = lax.fori_loop(0, _NPOINT, step, init)
    ids_ref[...] = ids
    scx_ref[...] = sx
    scy_ref[...] = sy
    scz_ref[...] = sz


def _run_fps(coorT):
    shape = jax.ShapeDtypeStruct((_B, _NPOINT), jnp.float32)
    ishape = jax.ShapeDtypeStruct((_B, _NPOINT), jnp.int32)
    return pl.pallas_call(
        _fps_body,
        out_shape=(ishape, shape, shape, shape),
        scratch_shapes=[pltpu.VMEM((_B, _N), jnp.float32)],
    )(coorT)


def _bq_body(packedT_ref, sc_ref, scT_ref, ids_ref,
             nidx_ref, diffc_ref, meanif_ref, sif_ref):
    b = pl.program_id(0)
    cxr = packedT_ref[0, 0:1, :]
    cyr = packedT_ref[0, 1:2, :]
    czr = packedT_ref[0, 2:3, :]
    scx = sc_ref[0, :, 0:1]
    scy = sc_ref[0, :, 1:2]
    scz = sc_ref[0, :, 2:3]
    dx = scx - cxr
    dy = scy - cyr
    dz = scz - czr
    d2 = (dx * dx + dy * dy) + dz * dz
    mask = d2 < _R2
    li = lax.broadcasted_iota(jnp.int32, (_NPOINT, _N), 1)
    cnt = jnp.sum(mask.astype(jnp.int32), axis=1, keepdims=True)
    mi0 = jnp.where(mask, li, _BIG)
    mi = mi0
    idxs = []
    for j in range(_NS):
        mn = jnp.min(mi, axis=1, keepdims=True)
        idxs.append(mn)
        mi = jnp.where(mi == mn, _BIG, mi)
    w = jnp.where(mi != mi0, 1.0, 0.0)
    first = jnp.where(cnt > 0, idxs[0], 0)
    pad = (_NS - jnp.minimum(cnt, _NS)).astype(jnp.float32)
    w = w + jnp.where(li == first, pad, 0.0)
    goff = b * _N
    for j in range(_NS):
        idx_j = jnp.where(j < cnt, idxs[j], first)
        nidx_ref[0, :, j:j + 1] = idx_j + goff
    eighth = jnp.float32(1.0 / _NS)
    p8 = packedT_ref[0]
    m6 = lax.dot_general(p8, w, (((1,), (1,)), ((), ())),
                         preferred_element_type=jnp.float32) * eighth
    diffc_ref[0, 0:3, :] = m6[0:3, :] - scT_ref[0]
    meanif_ref[0, 0:3, :] = m6[3:6, :]
    ws = (li == (ids_ref[0] - goff)).astype(jnp.float32)
    s6 = lax.dot_general(p8, ws, (((1,), (1,)), ((), ())),
                         preferred_element_type=jnp.float32)
    sif_ref[0, 0:3, :] = s6[3:6, :]


def _run_bq(packedT, sample_coor, scT, ids_col):
    spec3c = pl.BlockSpec((1, 3, _NPOINT), lambda b: (b, 0, 0))
    return pl.pallas_call(
        _bq_body,
        grid=(_B,),
        in_specs=[
            pl.BlockSpec((1, 8, _N), lambda b: (b, 0, 0)),
            pl.BlockSpec((1, _NPOINT, 3), lambda b: (b, 0, 0)),
            spec3c,
            pl.BlockSpec((1, _NPOINT, 1), lambda b: (b, 0, 0)),
        ],
        out_specs=[
            pl.BlockSpec((1, _NPOINT, _NS), lambda b: (b, 0, 0)),
            spec3c, spec3c, spec3c,
        ],
        out_shape=[
            jax.ShapeDtypeStruct((_B, _NPOINT, _NS), jnp.int32),
            jax.ShapeDtypeStruct((_B, 3, _NPOINT), jnp.float32),
            jax.ShapeDtypeStruct((_B, 3, _NPOINT), jnp.float32),
            jax.ShapeDtypeStruct((_B, 3, _NPOINT), jnp.float32),
        ],
    )(packedT, sample_coor, scT, ids_col)


_NWORK = 32
_S_PER_W = (_B * _NPOINT) // _NWORK
_ROWS_PER_W = _S_PER_W * _NS


def _sc_body(x_hbm, sidx_hbm, nidx_hbm, sx_out, gx_out,
             sidx_v, nidx_v, srows, nrows, pooled, sem1, sem2):
    wid = lax.axis_index("s") * 2 + lax.axis_index("c")
    sb = wid * _S_PER_W
    nb = wid * _ROWS_PER_W
    pltpu.sync_copy(sidx_hbm.at[pl.ds(sb, _S_PER_W)], sidx_v)
    pltpu.sync_copy(nidx_hbm.at[pl.ds(nb, _ROWS_PER_W)], nidx_v)
    c1 = pltpu.async_copy(x_hbm.at[nidx_v], nrows, sem1)
    c2 = pltpu.async_copy(x_hbm.at[sidx_v], srows, sem2)
    c1.wait()

    def pool_one(s, carry):
        base = s * _NS
        for c in range(_DIM // 16):
            sl = pl.ds(c * 16, 16)
            m = nrows[base, sl]
            for r in range(1, _NS):
                m = jnp.maximum(m, nrows[base + r, sl])
            pooled[s, sl] = m
        return carry

    lax.fori_loop(0, _S_PER_W, pool_one, 0)
    c2.wait()
    pltpu.sync_copy(pooled, gx_out.at[pl.ds(sb, _S_PER_W)])
    pltpu.sync_copy(srows, sx_out.at[pl.ds(sb, _S_PER_W)])


def _run_gather_pool(x2d, sidx, nidx):
    nsamp = _B * _NPOINT
    mesh = plsc.VectorSubcoreMesh(core_axis_name="c", subcore_axis_name="s")
    f = pl.kernel(
        _sc_body,
        out_type=(
            jax.ShapeDtypeStruct((nsamp, _DIM), jnp.float32),
            jax.ShapeDtypeStruct((nsamp, _DIM), jnp.float32),
        ),
        mesh=mesh,
        scratch_types=[
            pltpu.VMEM((_S_PER_W,), jnp.int32),
            pltpu.VMEM((_ROWS_PER_W,), jnp.int32),
            pltpu.VMEM((_S_PER_W, _DIM), jnp.float32),
            pltpu.VMEM((_ROWS_PER_W, _DIM), jnp.float32),
            pltpu.VMEM((_S_PER_W, _DIM), jnp.float32),
            pltpu.SemaphoreType.DMA,
            pltpu.SemaphoreType.DMA,
        ],
    )
    return f(x2d, sidx, nidx)


def _ln(v, g, bvec):
    mu = jnp.mean(v, axis=-1, keepdims=True)
    var = jnp.mean((v - mu) ** 2, axis=-1, keepdims=True)
    return (v - mu) / jnp.sqrt(var + 1e-5) * g + bvec


def _attn_body(sx_ref, gx_ref, vc_ref, vi_ref, sc_ref, sif_ref,
               wqt_ref, wkt_ref, gq_ref, bq_ref, gk_ref, bk_ref,
               outx_ref, outc_ref, outi_ref):
    sxb = sx_ref[0]
    gxb = gx_ref[0]
    x2 = gxb - sxb
    nk = _ln(sxb, gk_ref[...], bk_ref[...])
    nq = _ln(x2, gq_ref[...], bq_ref[...])
    qh = jnp.dot(nq, wqt_ref[...], preferred_element_type=jnp.float32)
    kh = jnp.dot(nk, wkt_ref[...], preferred_element_type=jnp.float32)
    attn = lax.dot_general(qh, kh, (((1,), (1,)), ((), ())),
                           preferred_element_type=jnp.float32)
    mx = jnp.max(attn, axis=-1, keepdims=True)
    e = jnp.exp(attn - mx)
    p = e / jnp.sum(e, axis=-1, keepdims=True)
    c2 = jnp.dot(p, vc_ref[0], preferred_element_type=jnp.float32)
    i2 = jnp.dot(p, vi_ref[0], preferred_element_type=jnp.float32)
    outx_ref[0] = sxb + x2
    outc_ref[0] = sc_ref[0] + c2
    outi_ref[0] = sif_ref[0] + i2


def _run_attn(sx, gx, v_c, v_i, sample_coor, sif, WqT, WkT, gq, bq, gk, bk):
    spec64 = pl.BlockSpec((1, _NPOINT, _DIM), lambda b: (b, 0, 0))
    spec3 = pl.BlockSpec((1, _NPOINT, 3), lambda b: (b, 0, 0))
    specw = pl.BlockSpec((_DIM, _DIM), lambda b: (0, 0))
    specv = pl.BlockSpec((1, _DIM), lambda b: (0, 0))
    return pl.pallas_call(
        _attn_body,
        grid=(_B,),
        in_specs=[spec64, spec64, spec3, spec3, spec3, spec3,
                  specw, specw, specv, specv, specv, specv],
        out_specs=[spec64, spec3, spec3],
        out_shape=[
            jax.ShapeDtypeStruct((_B, _NPOINT, _DIM), jnp.float32),
            jax.ShapeDtypeStruct((_B, _NPOINT, 3), jnp.float32),
            jax.ShapeDtypeStruct((_B, _NPOINT, 3), jnp.float32),
        ],
    )(sx, gx, v_c, v_i, sample_coor, sif, WqT, WkT, gq, bq, gk, bk)


def kernel(input_feature, x, coor, Wq, Wk, gq, bq, gk, bk):
    coorT = jnp.transpose(coor, (0, 2, 1))
    ifT = jnp.transpose(input_feature, (0, 2, 1))
    packedT = jnp.concatenate(
        [coorT, ifT, jnp.zeros((_B, 2, _N), jnp.float32)], axis=1)
    ids_g, scx, scy, scz = _run_fps(coorT)
    sample_coor = jnp.stack([scx, scy, scz], axis=-1)
    scT = jnp.stack([scx, scy, scz], axis=1)
    nidx, diffc, meanif, sif3 = _run_bq(
        packedT, sample_coor, scT, ids_g.reshape(_B, _NPOINT, 1))
    sx, gx = _run_gather_pool(
        x.reshape(_B * _N, _DIM), ids_g.reshape(-1), nidx.reshape(-1))
    v_c = diffc.reshape(_B, _NPOINT, 3)
    v_i = meanif.reshape(_B, _NPOINT, 3)
    sif = jnp.transpose(sif3, (0, 2, 1))
    return _run_attn(
        sx.reshape(_B, _NPOINT, _DIM), gx.reshape(_B, _NPOINT, _DIM),
        v_c, v_i, sample_coor, sif, Wq.T, Wk.T,
        gq.reshape(1, _DIM), bq.reshape(1, _DIM),
        gk.reshape(1, _DIM), bk.reshape(1, _DIM))
